# baseline (speedup 1.0000x reference)
import jax
import jax.numpy as jnp
from jax.experimental import pallas as pl
from jax.experimental.pallas import tpu as pltpu


def _slice_copy_kernel(w_ref, o_ref, sem):
    seq_len = o_ref.shape[0]
    copy = pltpu.make_async_copy(w_ref.at[pl.ds(0, seq_len)], o_ref, sem)
    copy.start()
    copy.wait()


def kernel(x, weight):
    seq_len = x.shape[-2]
    return pl.pallas_call(
        _slice_copy_kernel,
        out_shape=jax.ShapeDtypeStruct((seq_len, weight.shape[1]), weight.dtype),
        in_specs=[pl.BlockSpec(memory_space=pl.ANY)],
        out_specs=pl.BlockSpec(memory_space=pl.ANY),
        scratch_shapes=[pltpu.SemaphoreType.DMA],
    )(weight)


# speedup vs baseline: 1.0004x; 1.0004x over previous
import jax
import jax.numpy as jnp
from jax.experimental import pallas as pl
from jax.experimental.pallas import tpu as pltpu


_NCHUNKS = 32


def _slice_copy_kernel(w_ref, o_ref, sems):
    seq_len = o_ref.shape[0]
    chunk = seq_len // _NCHUNKS
    copies = [
        pltpu.make_async_copy(
            w_ref.at[pl.ds(i * chunk, chunk)],
            o_ref.at[pl.ds(i * chunk, chunk)],
            sems.at[i],
        )
        for i in range(_NCHUNKS)
    ]
    for c in copies:
        c.start()
    for c in copies:
        c.wait()


def kernel(x, weight):
    seq_len = x.shape[-2]
    return pl.pallas_call(
        _slice_copy_kernel,
        out_shape=jax.ShapeDtypeStruct((seq_len, weight.shape[1]), weight.dtype),
        in_specs=[pl.BlockSpec(memory_space=pl.ANY)],
        out_specs=pl.BlockSpec(memory_space=pl.ANY),
        scratch_shapes=[pltpu.SemaphoreType.DMA((_NCHUNKS,))],
    )(weight)


# speedup vs baseline: 46.9062x; 46.8862x over previous
import jax
import jax.numpy as jnp
from jax.experimental import pallas as pl
from jax.experimental.pallas import tpu as pltpu


_BLOCK_ROWS = 512


def _slice_copy_kernel(w_ref, o_ref):
    o_ref[...] = w_ref[...]


def kernel(x, weight):
    seq_len = x.shape[-2]
    dim = weight.shape[1]
    grid = (seq_len // _BLOCK_ROWS,)
    return pl.pallas_call(
        _slice_copy_kernel,
        out_shape=jax.ShapeDtypeStruct((seq_len, dim), weight.dtype),
        grid=grid,
        in_specs=[pl.BlockSpec((_BLOCK_ROWS, dim), lambda i: (i, 0))],
        out_specs=pl.BlockSpec((_BLOCK_ROWS, dim), lambda i: (i, 0)),
    )(weight)
